# baseline (speedup 1.0000x reference)
import functools

import jax
import jax.numpy as jnp
from jax import lax
from jax.experimental import pallas as pl
from jax.experimental.pallas import tpu as pltpu
from jax.experimental.pallas import tpu_sc as plsc

N = 1000000
DIM = 32
B = 16384
NC = 2
NS = 16
NW = NC * NS
BPW = B // NW
CHUNK = 128
KCH = BPW // CHUNK
HALVES = 2
CPH = KCH // HALVES
RPH = BPW // HALVES

_LN2 = 0.6931471805599453


def _log_sigmoid(x):
    e = jnp.exp(-x)
    y = 1.0 / (1.0 + e) + 1e-20
    i = lax.bitcast_convert_type(y, jnp.int32)
    ix = i + jnp.int32(0x3F800000 - 0x3F3504F3)
    k = lax.shift_right_arithmetic(ix, jnp.int32(23)) - jnp.int32(127)
    mb = lax.bitwise_and(ix, jnp.int32(0x007FFFFF)) + jnp.int32(0x3F3504F3)
    m = lax.bitcast_convert_type(mb, jnp.float32)
    f = m - 1.0
    s = f / (2.0 + f)
    z = s * s
    w = z * z
    t1 = w * (0.40000972152 + w * 0.24279078841)
    t2 = z * (0.66666662693 + w * 0.28498786688)
    r = t2 + t1
    hfsq = 0.5 * f * f
    kf = k.astype(jnp.float32)
    return kf * _LN2 + (f - (hfsq - s * (hfsq + r)))


@functools.lru_cache(maxsize=1)
def _build_sc_kernel():
    mesh = plsc.VectorSubcoreMesh(core_axis_name="c", subcore_axis_name="s")

    @functools.partial(
        pl.kernel,
        mesh=mesh,
        compiler_params=pltpu.CompilerParams(needs_layout_passes=False),
        out_type=jax.ShapeDtypeStruct((B,), jnp.float32),
        scratch_types=[
            pltpu.VMEM((KCH, CHUNK), jnp.int32),
            pltpu.VMEM((KCH, CHUNK), jnp.int32),
            pltpu.VMEM((BPW,), jnp.int32),
            pltpu.VMEM((BPW,), jnp.int32),
            pltpu.VMEM((RPH, 128), jnp.float32),
            pltpu.VMEM((RPH, 128), jnp.float32),
            pltpu.VMEM((BPW,), jnp.float32),
            pltpu.SemaphoreType.DMA,
            pltpu.SemaphoreType.DMA,
        ],
    )
    def sc_kernel(q1_hbm, q2_hbm, r1_hbm, r2_hbm, t1_hbm, t2_hbm, out_hbm,
                  q1_v, q2_v, r1_v, r2_v, rows1, rows2, out_v, sem1, sem2):
        wid = lax.axis_index("s") * NC + lax.axis_index("c")
        base = wid * BPW

        pltpu.sync_copy(q1_hbm.at[wid], q1_v)
        pltpu.sync_copy(q2_hbm.at[wid], q2_v)
        pltpu.sync_copy(r1_hbm.at[wid], r1_v)
        pltpu.sync_copy(r2_hbm.at[wid], r2_v)

        lane = lax.iota(jnp.int32, 16)

        for h in range(HALVES):
            copies = []
            for j in range(CPH):
                jj = h * CPH + j
                dst1 = rows1.at[pl.ds(j * CHUNK, CHUNK)]
                dst2 = rows2.at[pl.ds(j * CHUNK, CHUNK)]
                copies.append(
                    pltpu.async_copy(t1_hbm.at[q1_v.at[jj]], dst1, sem1))
                copies.append(
                    pltpu.async_copy(t2_hbm.at[q2_v.at[jj]], dst2, sem2))
            for c in copies:
                c.wait()

            def group_body(g, carry, h=h):
                row_idx = lane + g * 16
                off = h * RPH + g * 16
                c1 = r1_v[pl.ds(off, 16)]
                c2 = r2_v[pl.ds(off, 16)]
                acc = jnp.zeros((16,), jnp.float32)
                for c in range(DIM):
                    acc = acc + plsc.load_gather(rows1, [row_idx, c1 + c]) * \
                        plsc.load_gather(rows2, [row_idx, c2 + c])
                out_v[pl.ds(off, 16)] = _log_sigmoid(acc)
                return carry

            lax.fori_loop(0, RPH // 16, group_body, 0)

        pltpu.sync_copy(out_v, out_hbm.at[pl.ds(base, BPW)])

    return sc_kernel


def kernel(idx1, idx2, emb1, emb2, embs1, embs2):
    del emb1, emb2
    i1 = idx1.astype(jnp.int32)
    i2 = idx2.astype(jnp.int32)
    q1 = lax.shift_right_logical(i1, 2).reshape(NW, KCH, CHUNK)
    q2 = lax.shift_right_logical(i2, 2).reshape(NW, KCH, CHUNK)
    r1 = (lax.bitwise_and(i1, 3) * 32).reshape(NW, BPW)
    r2 = (lax.bitwise_and(i2, 3) * 32).reshape(NW, BPW)
    t1 = embs1.reshape(N // 4, 4 * DIM)
    t2 = embs2.reshape(N // 4, 4 * DIM)
    return _build_sc_kernel()(q1, q2, r1, r2, t1, t2)


# speedup vs baseline: 3.5437x; 3.5437x over previous
import functools

import jax
import jax.numpy as jnp
from jax import lax
from jax.experimental import pallas as pl
from jax.experimental.pallas import tpu as pltpu
from jax.experimental.pallas import tpu_sc as plsc

N = 1000000
DIM = 32
B = 16384
NC = 2
NS = 16
NW = NC * NS
BPW = B // NW

WL = 1024
NWIN_FULL = N // WL
TAIL0 = NWIN_FULL * WL
TAILA = 512
TAILB = N - TAIL0 - TAILA
WPT = -(-(NWIN_FULL) // NW)
MYCAP = 1024
HITCAP = 256
NSLOT = 16
LAG = 12

_SENTINEL = 0x7FFFFFF0
_LN2 = 0.6931471805599453


def _log_sigmoid(x):
    e = jnp.exp(-x)
    y = 1.0 / (1.0 + e) + 1e-20
    i = lax.bitcast_convert_type(y, jnp.int32)
    ix = i + jnp.int32(0x3F800000 - 0x3F3504F3)
    k = lax.shift_right_arithmetic(ix, jnp.int32(23)) - jnp.int32(127)
    mb = lax.bitwise_and(ix, jnp.int32(0x007FFFFF)) + jnp.int32(0x3F3504F3)
    m = lax.bitcast_convert_type(mb, jnp.float32)
    f = m - 1.0
    s = f / (2.0 + f)
    z = s * s
    w = z * z
    t1 = w * (0.40000972152 + w * 0.24279078841)
    t2 = z * (0.66666662693 + w * 0.28498786688)
    r = t2 + t1
    hfsq = 0.5 * f * f
    kf = k.astype(jnp.float32)
    return kf * _LN2 + (f - (hfsq - s * (hfsq + r)))


def _mesh():
    return plsc.VectorSubcoreMesh(core_axis_name="c", subcore_axis_name="s")


@functools.lru_cache(maxsize=1)
def _build_phase1():
    @functools.partial(
        pl.kernel,
        mesh=_mesh(),
        compiler_params=pltpu.CompilerParams(needs_layout_passes=False),
        out_type=(
            jax.ShapeDtypeStruct((B * DIM,), jnp.float32),
            jax.ShapeDtypeStruct((B * DIM,), jnp.float32),
        ),
        scratch_types=[
            pltpu.VMEM((B,), jnp.int32),
            pltpu.VMEM((B,), jnp.int32),
            pltpu.VMEM((MYCAP,), jnp.int32),
            pltpu.VMEM((MYCAP,), jnp.int32),
            pltpu.VMEM((MYCAP,), jnp.int32),
            pltpu.VMEM((MYCAP,), jnp.int32),
            pltpu.VMEM((HITCAP,), jnp.int32),
            pltpu.VMEM((HITCAP,), jnp.int32),
            pltpu.VMEM((4, 8, WL), jnp.float32),
            pltpu.VMEM((4, 8, TAILA), jnp.float32),
            pltpu.VMEM((DIM, TAILB), jnp.float32),
            pltpu.VMEM((NSLOT, DIM), jnp.float32),
            pltpu.SemaphoreType.DMA,
            pltpu.SemaphoreType.DMA,
        ],
    )
    def phase1(idx1_hbm, idx2_hbm, t1_hbm, t2_hbm, tb1_hbm, tb2_hbm,
               e1_hbm, e2_hbm,
               idx1_v, idx2_v, myk0, myk1, myi0, myi1, hitk, hitc,
               slab, tslab_a, tslab_b, rowstage, sem_s, sem_w):
        wid = lax.axis_index("s") * NC + lax.axis_index("c")
        lane = lax.iota(jnp.int32, 16)
        b0 = lax.shift_right_logical(lane, jnp.int32(3))
        s0 = lax.bitwise_and(lane, jnp.int32(7))
        b1 = b0 + 2

        pltpu.sync_copy(idx1_hbm, idx1_v)
        pltpu.sync_copy(idx2_hbm, idx2_v)

        sent = jnp.full((16,), _SENTINEL, jnp.int32)
        def fill_body(g, carry):
            myi0[pl.ds(g * 16, 16)] = sent
            myi1[pl.ds(g * 16, 16)] = sent
            return carry
        lax.fori_loop(0, MYCAP // 16, fill_body, 0)

        for t in range(2):
            idxv = idx1_v if t == 0 else idx2_v
            mk = myk0 if t == 0 else myk1
            mi = myi0 if t == 0 else myi1

            def extract_body(g, cur, idxv=idxv, mk=mk, mi=mi):
                iv = idxv[pl.ds(g * 16, 16)]
                w = lax.shift_right_logical(iv, jnp.int32(10))
                m = lax.bitwise_and(w, jnp.int32(NW - 1)) == wid
                kv = lane + g * 16
                plsc.store_compressed(mk.at[pl.ds(cur, 16)], kv, mask=m)
                plsc.store_compressed(mi.at[pl.ds(cur, 16)], iv, mask=m)
                cnt = plsc.all_reduce_population_count(m)
                return cur + cnt[0]

            lax.fori_loop(0, B // 16, extract_body, jnp.int32(0))

        def find_hits(t, w, w0):
            mk = myk0 if t == 0 else myk1
            mi = myi0 if t == 0 else myi1

            def scan_body(g, cur):
                iv = mi[pl.ds(g * 16, 16)]
                m = lax.shift_right_logical(iv, jnp.int32(10)) == w
                kv = mk[pl.ds(g * 16, 16)]
                plsc.store_compressed(hitk.at[pl.ds(cur, 16)], kv, mask=m)
                plsc.store_compressed(hitc.at[pl.ds(cur, 16)], iv - w0, mask=m)
                cnt = plsc.all_reduce_population_count(m)
                return cur + cnt[0]
            return lax.fori_loop(0, MYCAP // 16, scan_body, jnp.int32(0))

        def write_hits(nh, eout, gather_row):
            def hit_body(h, carry):
                kvec = plsc.load_gather(hitk, [jnp.full((16,), h, jnp.int32)])
                cvec = plsc.load_gather(hitc, [jnp.full((16,), h, jnp.int32)])
                k = kvec[0]
                v0, v1 = gather_row(cvec)
                slot = lax.rem(h, jnp.int32(NSLOT))
                rowstage[slot, pl.ds(0, 16)] = v0
                rowstage[slot, pl.ds(16, 16)] = v1
                pltpu.async_copy(rowstage.at[slot],
                                 eout.at[pl.ds(k * DIM, DIM)], sem_w)

                @pl.when(h >= LAG)
                def _():
                    pltpu.make_async_copy(
                        eout.at[pl.ds(0, DIM)], rowstage.at[0], sem_w).wait()

                return carry

            lax.fori_loop(0, nh, hit_body, 0)

            def drain_body(r, carry):
                pltpu.make_async_copy(
                    eout.at[pl.ds(0, DIM)], rowstage.at[0], sem_w).wait()
                return carry

            lax.fori_loop(0, jnp.minimum(nh, jnp.int32(LAG)), drain_body, 0)

        for t in range(2):
            tbl = t1_hbm if t == 0 else t2_hbm
            eout = e1_hbm if t == 0 else e2_hbm

            def win_body(n, carry, t=t, tbl=tbl, eout=eout):
                w = wid + n * NW

                @pl.when(w < NWIN_FULL)
                def _():
                    w0 = w * WL
                    for b in range(4):
                        pltpu.async_copy(
                            tbl.at[b, :, pl.ds(w0, WL)], slab.at[b], sem_s)
                    nh = find_hits(t, w, w0)
                    for b in range(4):
                        pltpu.make_async_copy(
                            tbl.at[b, :, pl.ds(0, WL)], slab.at[b],
                            sem_s).wait()

                    def gather_row(cvec):
                        return (plsc.load_gather(slab, [b0, s0, cvec]),
                                plsc.load_gather(slab, [b1, s0, cvec]))

                    write_hits(nh, eout, gather_row)

                return carry

            lax.fori_loop(0, WPT, win_body, 0)

        for t in range(2):
            tbl = t1_hbm if t == 0 else t2_hbm
            tbt = tb1_hbm if t == 0 else tb2_hbm
            eout = e1_hbm if t == 0 else e2_hbm
            idxv = idx1_v if t == 0 else idx2_v
            for b in range(4):
                pltpu.async_copy(
                    tbl.at[b, :, pl.ds(TAIL0, TAILA)], tslab_a.at[b], sem_s)
            pltpu.sync_copy(tbt, tslab_b)

            def tail_scan(g, cur, idxv=idxv):
                iv = idxv[pl.ds(g * 16, 16)]
                m = iv >= TAIL0
                kv = lane + g * 16
                plsc.store_compressed(hitk.at[pl.ds(cur, 16)], kv, mask=m)
                plsc.store_compressed(hitc.at[pl.ds(cur, 16)], iv - TAIL0, mask=m)
                cnt = plsc.all_reduce_population_count(m)
                return cur + cnt[0]

            nh = lax.fori_loop(0, B // 16, tail_scan, jnp.int32(0))
            for b in range(4):
                pltpu.make_async_copy(
                    tbl.at[b, :, pl.ds(0, TAILA)], tslab_a.at[b], sem_s).wait()

            def gather_row_tail(cvec):
                col = cvec[0]

                def in_a():
                    return (plsc.load_gather(tslab_a, [b0, s0, cvec]),
                            plsc.load_gather(tslab_a, [b1, s0, cvec]))

                def in_b():
                    cb = cvec - TAILA
                    lane16 = b0 * 8 + s0
                    return (plsc.load_gather(tslab_b, [lane16, cb]),
                            plsc.load_gather(tslab_b, [lane16 + 16, cb]))

                return lax.cond(col < TAILA, in_a, in_b)

            write_hits(nh, eout, gather_row_tail)

    return phase1


@functools.lru_cache(maxsize=1)
def _build_phase2():
    @functools.partial(
        pl.kernel,
        mesh=_mesh(),
        compiler_params=pltpu.CompilerParams(needs_layout_passes=False),
        out_type=jax.ShapeDtypeStruct((B,), jnp.float32),
        scratch_types=[
            pltpu.VMEM((BPW * DIM,), jnp.float32),
            pltpu.VMEM((BPW * DIM,), jnp.float32),
            pltpu.VMEM((BPW,), jnp.float32),
        ],
    )
    def phase2(e1_hbm, e2_hbm, out_hbm, e1_v, e2_v, out_v):
        wid = lax.axis_index("s") * NC + lax.axis_index("c")
        base = wid * BPW
        pltpu.sync_copy(e1_hbm.at[pl.ds(base * DIM, BPW * DIM)], e1_v)
        pltpu.sync_copy(e2_hbm.at[pl.ds(base * DIM, BPW * DIM)], e2_v)
        lane32 = lax.iota(jnp.int32, 16) * DIM

        def group_body(g, carry):
            off0 = g * (16 * DIM)
            acc = jnp.zeros((16,), jnp.float32)
            for c in range(DIM):
                offs = lane32 + (off0 + c)
                acc = acc + plsc.load_gather(e1_v, [offs]) * \
                    plsc.load_gather(e2_v, [offs])
            out_v[pl.ds(g * 16, 16)] = _log_sigmoid(acc)
            return carry

        lax.fori_loop(0, BPW // 16, group_body, 0)
        pltpu.sync_copy(out_v, out_hbm.at[pl.ds(base, BPW)])

    return phase2


def kernel(idx1, idx2, emb1, emb2, embs1, embs2):
    del emb1, emb2
    i1 = idx1.astype(jnp.int32)
    i2 = idx2.astype(jnp.int32)
    t1 = embs1.T.reshape(4, 8, N)
    t2 = embs2.T.reshape(4, 8, N)
    tb1 = embs1[TAIL0 + TAILA:].T
    tb2 = embs2[TAIL0 + TAILA:].T
    e1all, e2all = _build_phase1()(i1, i2, t1, t2, tb1, tb2)
    return _build_phase2()(e1all, e2all)


# speedup vs baseline: 4.8303x; 1.3631x over previous
import functools

import jax
import jax.numpy as jnp
from jax import lax
from jax.experimental import pallas as pl
from jax.experimental.pallas import tpu as pltpu
from jax.experimental.pallas import tpu_sc as plsc

N = 1000000
DIM = 32
B = 16384
NC = 2
NS = 16
NW = NC * NS
BPW = B // NW

WL = 1024
NWIN_FULL = N // WL
TAIL0 = NWIN_FULL * WL
TAILA = 512
TAILB = N - TAIL0 - TAILA
WPT = -(-(NWIN_FULL) // NW)
MYCAP = 1024
HITCAP = 256
NSLOT = 16
LAG = 12

_SENTINEL = 0x7FFFFFF0
_LN2 = 0.6931471805599453


def _log_sigmoid(x):
    e = jnp.exp(-x)
    y = 1.0 / (1.0 + e) + 1e-20
    i = lax.bitcast_convert_type(y, jnp.int32)
    ix = i + jnp.int32(0x3F800000 - 0x3F3504F3)
    k = lax.shift_right_arithmetic(ix, jnp.int32(23)) - jnp.int32(127)
    mb = lax.bitwise_and(ix, jnp.int32(0x007FFFFF)) + jnp.int32(0x3F3504F3)
    m = lax.bitcast_convert_type(mb, jnp.float32)
    f = m - 1.0
    s = f / (2.0 + f)
    z = s * s
    w = z * z
    t1 = w * (0.40000972152 + w * 0.24279078841)
    t2 = z * (0.66666662693 + w * 0.28498786688)
    r = t2 + t1
    hfsq = 0.5 * f * f
    kf = k.astype(jnp.float32)
    return kf * _LN2 + (f - (hfsq - s * (hfsq + r)))


def _mesh():
    return plsc.VectorSubcoreMesh(core_axis_name="c", subcore_axis_name="s")


@functools.lru_cache(maxsize=1)
def _build_phase1():
    @functools.partial(
        pl.kernel,
        mesh=_mesh(),
        compiler_params=pltpu.CompilerParams(needs_layout_passes=False),
        out_type=(
            jax.ShapeDtypeStruct((B * DIM,), jnp.float32),
            jax.ShapeDtypeStruct((B * DIM,), jnp.float32),
        ),
        scratch_types=[
            pltpu.VMEM((B,), jnp.int32),
            pltpu.VMEM((B,), jnp.int32),
            pltpu.VMEM((MYCAP,), jnp.int32),
            pltpu.VMEM((MYCAP,), jnp.int32),
            pltpu.VMEM((MYCAP,), jnp.int32),
            pltpu.VMEM((MYCAP,), jnp.int32),
            pltpu.VMEM((HITCAP,), jnp.int32),
            pltpu.VMEM((HITCAP,), jnp.int32),
            pltpu.VMEM((2, 4, 8, WL), jnp.float32),
            pltpu.VMEM((4, 8, TAILA), jnp.float32),
            pltpu.VMEM((DIM, TAILB), jnp.float32),
            pltpu.VMEM((NSLOT, DIM), jnp.float32),
            pltpu.SemaphoreType.DMA,
            pltpu.SemaphoreType.DMA,
        ],
    )
    def phase1(idx1_hbm, idx2_hbm, t1_hbm, t2_hbm, tb1_hbm, tb2_hbm,
               e1_hbm, e2_hbm,
               idx1_v, idx2_v, myk0, myk1, myi0, myi1, hitk, hitc,
               slab, tslab_a, tslab_b, rowstage, sem_s, sem_w):
        wid = lax.axis_index("s") * NC + lax.axis_index("c")
        lane = lax.iota(jnp.int32, 16)
        b0 = lax.shift_right_logical(lane, jnp.int32(3))
        s0 = lax.bitwise_and(lane, jnp.int32(7))
        b1 = b0 + 2

        pltpu.sync_copy(idx1_hbm, idx1_v)
        pltpu.sync_copy(idx2_hbm, idx2_v)

        sent = jnp.full((16,), _SENTINEL, jnp.int32)
        def fill_body(g, carry):
            myi0[pl.ds(g * 16, 16)] = sent
            myi1[pl.ds(g * 16, 16)] = sent
            return carry
        lax.fori_loop(0, MYCAP // 16, fill_body, 0)

        nmine = []
        for t in range(2):
            idxv = idx1_v if t == 0 else idx2_v
            mk = myk0 if t == 0 else myk1
            mi = myi0 if t == 0 else myi1

            def extract_body(g, cur, idxv=idxv, mk=mk, mi=mi):
                iv = idxv[pl.ds(g * 16, 16)]
                w = lax.shift_right_logical(iv, jnp.int32(10))
                m = lax.bitwise_and(w, jnp.int32(NW - 1)) == wid
                kv = lane + g * 16
                plsc.store_compressed(mk.at[pl.ds(cur, 16)], kv, mask=m)
                plsc.store_compressed(mi.at[pl.ds(cur, 16)], iv, mask=m)
                cnt = plsc.all_reduce_population_count(m)
                return cur + cnt[0]

            nmine.append(
                lax.fori_loop(0, B // 16, extract_body, jnp.int32(0)))

        def find_hits(t, w, w0):
            mk = myk0 if t == 0 else myk1
            mi = myi0 if t == 0 else myi1
            ng = lax.shift_right_logical(nmine[t] + 15, jnp.int32(4))

            def scan_body(g, cur):
                iv = mi[pl.ds(g * 16, 16)]
                m = lax.shift_right_logical(iv, jnp.int32(10)) == w
                kv = mk[pl.ds(g * 16, 16)]
                plsc.store_compressed(hitk.at[pl.ds(cur, 16)], kv, mask=m)
                plsc.store_compressed(hitc.at[pl.ds(cur, 16)], iv - w0, mask=m)
                cnt = plsc.all_reduce_population_count(m)
                return cur + cnt[0]
            return lax.fori_loop(0, ng, scan_body, jnp.int32(0))

        def write_hits(nh, eout, gather_row):
            def hit_body(h, carry):
                kvec = plsc.load_gather(hitk, [jnp.full((16,), h, jnp.int32)])
                cvec = plsc.load_gather(hitc, [jnp.full((16,), h, jnp.int32)])
                k = kvec[0]
                v0, v1 = gather_row(cvec)
                slot = lax.rem(h, jnp.int32(NSLOT))
                rowstage[slot, pl.ds(0, 16)] = v0
                rowstage[slot, pl.ds(16, 16)] = v1
                pltpu.async_copy(rowstage.at[slot],
                                 eout.at[pl.ds(k * DIM, DIM)], sem_w)

                @pl.when(h >= LAG)
                def _():
                    pltpu.make_async_copy(
                        eout.at[pl.ds(0, DIM)], rowstage.at[0], sem_w).wait()

                return carry

            lax.fori_loop(0, nh, hit_body, 0)

            def drain_body(r, carry):
                pltpu.make_async_copy(
                    eout.at[pl.ds(0, DIM)], rowstage.at[0], sem_w).wait()
                return carry

            lax.fori_loop(0, jnp.minimum(nh, jnp.int32(LAG)), drain_body, 0)

        for t in range(2):
            tbl = t1_hbm if t == 0 else t2_hbm
            eout = e1_hbm if t == 0 else e2_hbm

            @pl.when(wid < NWIN_FULL)
            def _(tbl=tbl):
                for b in range(4):
                    pltpu.async_copy(
                        tbl.at[b, :, pl.ds(wid * WL, WL)], slab.at[0, b],
                        sem_s)

            def win_body(n, carry, t=t, tbl=tbl, eout=eout):
                w = wid + n * NW
                buf = lax.rem(n, jnp.int32(2))
                nbuf = lax.rem(n + 1, jnp.int32(2))
                wn = w + NW

                @pl.when(wn < NWIN_FULL)
                def _():
                    for b in range(4):
                        pltpu.async_copy(
                            tbl.at[b, :, pl.ds(wn * WL, WL)],
                            slab.at[nbuf, b], sem_s)

                @pl.when(w < NWIN_FULL)
                def _():
                    w0 = w * WL
                    nh = find_hits(t, w, w0)
                    for b in range(4):
                        pltpu.make_async_copy(
                            tbl.at[b, :, pl.ds(0, WL)], slab.at[0, b],
                            sem_s).wait()
                    bufv = jnp.full((16,), buf, jnp.int32)

                    def gather_row(cvec):
                        return (plsc.load_gather(slab, [bufv, b0, s0, cvec]),
                                plsc.load_gather(slab, [bufv, b1, s0, cvec]))

                    write_hits(nh, eout, gather_row)

                return carry

            lax.fori_loop(0, WPT, win_body, 0)

        for t in range(2):
            tbl = t1_hbm if t == 0 else t2_hbm
            tbt = tb1_hbm if t == 0 else tb2_hbm
            eout = e1_hbm if t == 0 else e2_hbm
            idxv = idx1_v if t == 0 else idx2_v
            for b in range(4):
                pltpu.async_copy(
                    tbl.at[b, :, pl.ds(TAIL0, TAILA)], tslab_a.at[b], sem_s)
            pltpu.sync_copy(tbt, tslab_b)

            def tail_scan(g, cur, idxv=idxv):
                iv = idxv[pl.ds(g * 16, 16)]
                m = iv >= TAIL0
                kv = lane + g * 16
                plsc.store_compressed(hitk.at[pl.ds(cur, 16)], kv, mask=m)
                plsc.store_compressed(hitc.at[pl.ds(cur, 16)], iv - TAIL0, mask=m)
                cnt = plsc.all_reduce_population_count(m)
                return cur + cnt[0]

            nh = lax.fori_loop(0, B // 16, tail_scan, jnp.int32(0))
            for b in range(4):
                pltpu.make_async_copy(
                    tbl.at[b, :, pl.ds(0, TAILA)], tslab_a.at[b], sem_s).wait()

            def gather_row_tail(cvec):
                col = cvec[0]

                def in_a():
                    return (plsc.load_gather(tslab_a, [b0, s0, cvec]),
                            plsc.load_gather(tslab_a, [b1, s0, cvec]))

                def in_b():
                    cb = cvec - TAILA
                    lane16 = b0 * 8 + s0
                    return (plsc.load_gather(tslab_b, [lane16, cb]),
                            plsc.load_gather(tslab_b, [lane16 + 16, cb]))

                return lax.cond(col < TAILA, in_a, in_b)

            write_hits(nh, eout, gather_row_tail)

    return phase1


@functools.lru_cache(maxsize=1)
def _build_phase2():
    @functools.partial(
        pl.kernel,
        mesh=_mesh(),
        compiler_params=pltpu.CompilerParams(needs_layout_passes=False),
        out_type=jax.ShapeDtypeStruct((B,), jnp.float32),
        scratch_types=[
            pltpu.VMEM((BPW * DIM,), jnp.float32),
            pltpu.VMEM((BPW * DIM,), jnp.float32),
            pltpu.VMEM((BPW,), jnp.float32),
        ],
    )
    def phase2(e1_hbm, e2_hbm, out_hbm, e1_v, e2_v, out_v):
        wid = lax.axis_index("s") * NC + lax.axis_index("c")
        base = wid * BPW
        pltpu.sync_copy(e1_hbm.at[pl.ds(base * DIM, BPW * DIM)], e1_v)
        pltpu.sync_copy(e2_hbm.at[pl.ds(base * DIM, BPW * DIM)], e2_v)
        lane32 = lax.iota(jnp.int32, 16) * DIM

        def group_body(g, carry):
            off0 = g * (16 * DIM)
            acc = jnp.zeros((16,), jnp.float32)
            for c in range(DIM):
                offs = lane32 + (off0 + c)
                acc = acc + plsc.load_gather(e1_v, [offs]) * \
                    plsc.load_gather(e2_v, [offs])
            out_v[pl.ds(g * 16, 16)] = _log_sigmoid(acc)
            return carry

        lax.fori_loop(0, BPW // 16, group_body, 0)
        pltpu.sync_copy(out_v, out_hbm.at[pl.ds(base, BPW)])

    return phase2


def kernel(idx1, idx2, emb1, emb2, embs1, embs2):
    del emb1, emb2
    i1 = idx1.astype(jnp.int32)
    i2 = idx2.astype(jnp.int32)
    t1 = embs1.T.reshape(4, 8, N)
    t2 = embs2.T.reshape(4, 8, N)
    tb1 = embs1[TAIL0 + TAILA:].T
    tb2 = embs2[TAIL0 + TAILA:].T
    e1all, e2all = _build_phase1()(i1, i2, t1, t2, tb1, tb2)
    return _build_phase2()(e1all, e2all)


# speedup vs baseline: 4.9455x; 1.0239x over previous
import functools

import jax
import jax.numpy as jnp
from jax import lax
from jax.experimental import pallas as pl
from jax.experimental.pallas import tpu as pltpu
from jax.experimental.pallas import tpu_sc as plsc

N = 1000000
DIM = 32
B = 16384
NC = 2
NS = 16
NW = NC * NS
BPW = B // NW

WL = 1024
NWIN_FULL = N // WL
TAIL0 = NWIN_FULL * WL
TAILA = 512
TAILB = N - TAIL0 - TAILA
WPT = -(-(NWIN_FULL) // NW)
MYCAP = 1024
HITCAP = 256
NSLOT = 16
LAG = 12

_SENTINEL = 0x7FFFFFF0
_LN2 = 0.6931471805599453


def _log_sigmoid(x):
    e = jnp.exp(-x)
    y = 1.0 / (1.0 + e) + 1e-20
    i = lax.bitcast_convert_type(y, jnp.int32)
    ix = i + jnp.int32(0x3F800000 - 0x3F3504F3)
    k = lax.shift_right_arithmetic(ix, jnp.int32(23)) - jnp.int32(127)
    mb = lax.bitwise_and(ix, jnp.int32(0x007FFFFF)) + jnp.int32(0x3F3504F3)
    m = lax.bitcast_convert_type(mb, jnp.float32)
    f = m - 1.0
    s = f / (2.0 + f)
    z = s * s
    w = z * z
    t1 = w * (0.40000972152 + w * 0.24279078841)
    t2 = z * (0.66666662693 + w * 0.28498786688)
    r = t2 + t1
    hfsq = 0.5 * f * f
    kf = k.astype(jnp.float32)
    return kf * _LN2 + (f - (hfsq - s * (hfsq + r)))


def _mesh():
    return plsc.VectorSubcoreMesh(core_axis_name="c", subcore_axis_name="s")


@functools.lru_cache(maxsize=1)
def _build_phase1():
    @functools.partial(
        pl.kernel,
        mesh=_mesh(),
        compiler_params=pltpu.CompilerParams(needs_layout_passes=False),
        out_type=(
            jax.ShapeDtypeStruct((B * DIM,), jnp.float32),
            jax.ShapeDtypeStruct((B * DIM,), jnp.float32),
        ),
        scratch_types=[
            pltpu.VMEM((B,), jnp.int32),
            pltpu.VMEM((B,), jnp.int32),
            pltpu.VMEM((MYCAP,), jnp.int32),
            pltpu.VMEM((MYCAP,), jnp.int32),
            pltpu.VMEM((MYCAP,), jnp.int32),
            pltpu.VMEM((MYCAP,), jnp.int32),
            pltpu.VMEM((HITCAP,), jnp.int32),
            pltpu.VMEM((HITCAP,), jnp.int32),
            pltpu.VMEM((2, 4, 8, WL), jnp.float32),
            pltpu.VMEM((4, 8, TAILA), jnp.float32),
            pltpu.VMEM((DIM, TAILB), jnp.float32),
            pltpu.VMEM((NSLOT, DIM), jnp.float32),
            pltpu.SemaphoreType.DMA,
            pltpu.SemaphoreType.DMA,
        ],
    )
    def phase1(idx1_hbm, idx2_hbm, t1_hbm, t2_hbm, tb1_hbm, tb2_hbm,
               e1_hbm, e2_hbm,
               idx1_v, idx2_v, myk0, myk1, myi0, myi1, hitk, hitc,
               slab, tslab_a, tslab_b, rowstage, sem_s, sem_w):
        wid = lax.axis_index("s") * NC + lax.axis_index("c")
        lane = lax.iota(jnp.int32, 16)
        b0 = lax.shift_right_logical(lane, jnp.int32(3))
        s0 = lax.bitwise_and(lane, jnp.int32(7))
        b1 = b0 + 2

        pltpu.sync_copy(idx1_hbm, idx1_v)
        pltpu.sync_copy(idx2_hbm, idx2_v)

        sent = jnp.full((16,), _SENTINEL, jnp.int32)
        def fill_body(g, carry):
            myi0[pl.ds(g * 16, 16)] = sent
            myi1[pl.ds(g * 16, 16)] = sent
            return carry
        lax.fori_loop(0, MYCAP // 16, fill_body, 0)

        def extract_body(g2, curs):
            cur0, cur1 = curs
            for u in range(2):
                g = g2 * 2 + u
                kv = lane + g * 16
                iv1 = idx1_v[pl.ds(g * 16, 16)]
                w1 = lax.shift_right_logical(iv1, jnp.int32(10))
                m1 = lax.bitwise_and(w1, jnp.int32(NW - 1)) == wid
                plsc.store_compressed(myk0.at[pl.ds(cur0, 16)], kv, mask=m1)
                plsc.store_compressed(myi0.at[pl.ds(cur0, 16)], iv1, mask=m1)
                cur0 = cur0 + plsc.all_reduce_population_count(m1)[0]
                iv2 = idx2_v[pl.ds(g * 16, 16)]
                w2 = lax.shift_right_logical(iv2, jnp.int32(10))
                m2 = lax.bitwise_and(w2, jnp.int32(NW - 1)) == wid
                plsc.store_compressed(myk1.at[pl.ds(cur1, 16)], kv, mask=m2)
                plsc.store_compressed(myi1.at[pl.ds(cur1, 16)], iv2, mask=m2)
                cur1 = cur1 + plsc.all_reduce_population_count(m2)[0]
            return cur0, cur1

        nmine = list(lax.fori_loop(
            0, B // 32, extract_body, (jnp.int32(0), jnp.int32(0))))

        def find_hits(t, w, w0):
            mk = myk0 if t == 0 else myk1
            mi = myi0 if t == 0 else myi1
            ng2 = lax.shift_right_logical(nmine[t] + 31, jnp.int32(5))

            def scan_body(g2, cur):
                for u in range(2):
                    g = g2 * 2 + u
                    iv = mi[pl.ds(g * 16, 16)]
                    m = lax.shift_right_logical(iv, jnp.int32(10)) == w
                    kv = mk[pl.ds(g * 16, 16)]
                    plsc.store_compressed(hitk.at[pl.ds(cur, 16)], kv, mask=m)
                    plsc.store_compressed(
                        hitc.at[pl.ds(cur, 16)], iv - w0, mask=m)
                    cur = cur + plsc.all_reduce_population_count(m)[0]
                return cur
            return lax.fori_loop(0, ng2, scan_body, jnp.int32(0))

        def write_hits(nh, eout, gather_row):
            def hit_body(h, carry):
                kvec = plsc.load_gather(hitk, [jnp.full((16,), h, jnp.int32)])
                cvec = plsc.load_gather(hitc, [jnp.full((16,), h, jnp.int32)])
                k = kvec[0]
                v0, v1 = gather_row(cvec)
                slot = lax.rem(h, jnp.int32(NSLOT))
                rowstage[slot, pl.ds(0, 16)] = v0
                rowstage[slot, pl.ds(16, 16)] = v1
                pltpu.async_copy(rowstage.at[slot],
                                 eout.at[pl.ds(k * DIM, DIM)], sem_w)

                @pl.when(h >= LAG)
                def _():
                    pltpu.make_async_copy(
                        eout.at[pl.ds(0, DIM)], rowstage.at[0], sem_w).wait()

                return carry

            lax.fori_loop(0, nh, hit_body, 0)

            def drain_body(r, carry):
                pltpu.make_async_copy(
                    eout.at[pl.ds(0, DIM)], rowstage.at[0], sem_w).wait()
                return carry

            lax.fori_loop(0, jnp.minimum(nh, jnp.int32(LAG)), drain_body, 0)

        for t in range(2):
            tbl = t1_hbm if t == 0 else t2_hbm
            eout = e1_hbm if t == 0 else e2_hbm

            @pl.when(wid < NWIN_FULL)
            def _(tbl=tbl):
                for b in range(4):
                    pltpu.async_copy(
                        tbl.at[b, :, pl.ds(wid * WL, WL)], slab.at[0, b],
                        sem_s)

            def win_body(n, carry, t=t, tbl=tbl, eout=eout):
                w = wid + n * NW
                buf = lax.rem(n, jnp.int32(2))
                nbuf = lax.rem(n + 1, jnp.int32(2))
                wn = w + NW

                @pl.when(wn < NWIN_FULL)
                def _():
                    for b in range(4):
                        pltpu.async_copy(
                            tbl.at[b, :, pl.ds(wn * WL, WL)],
                            slab.at[nbuf, b], sem_s)

                @pl.when(w < NWIN_FULL)
                def _():
                    w0 = w * WL
                    nh = find_hits(t, w, w0)
                    for b in range(4):
                        pltpu.make_async_copy(
                            tbl.at[b, :, pl.ds(0, WL)], slab.at[0, b],
                            sem_s).wait()
                    bufv = jnp.full((16,), buf, jnp.int32)

                    def gather_row(cvec):
                        return (plsc.load_gather(slab, [bufv, b0, s0, cvec]),
                                plsc.load_gather(slab, [bufv, b1, s0, cvec]))

                    write_hits(nh, eout, gather_row)

                return carry

            lax.fori_loop(0, WPT, win_body, 0)

        for t in range(2):
            tbl = t1_hbm if t == 0 else t2_hbm
            tbt = tb1_hbm if t == 0 else tb2_hbm
            eout = e1_hbm if t == 0 else e2_hbm
            idxv = idx1_v if t == 0 else idx2_v
            for b in range(4):
                pltpu.async_copy(
                    tbl.at[b, :, pl.ds(TAIL0, TAILA)], tslab_a.at[b], sem_s)
            pltpu.sync_copy(tbt, tslab_b)

            def tail_scan(g, cur, idxv=idxv):
                iv = idxv[pl.ds(g * 16, 16)]
                m = iv >= TAIL0
                kv = lane + g * 16
                plsc.store_compressed(hitk.at[pl.ds(cur, 16)], kv, mask=m)
                plsc.store_compressed(hitc.at[pl.ds(cur, 16)], iv - TAIL0, mask=m)
                cnt = plsc.all_reduce_population_count(m)
                return cur + cnt[0]

            nh = lax.fori_loop(0, B // 16, tail_scan, jnp.int32(0))
            for b in range(4):
                pltpu.make_async_copy(
                    tbl.at[b, :, pl.ds(0, TAILA)], tslab_a.at[b], sem_s).wait()

            def gather_row_tail(cvec):
                col = cvec[0]

                def in_a():
                    return (plsc.load_gather(tslab_a, [b0, s0, cvec]),
                            plsc.load_gather(tslab_a, [b1, s0, cvec]))

                def in_b():
                    cb = cvec - TAILA
                    lane16 = b0 * 8 + s0
                    return (plsc.load_gather(tslab_b, [lane16, cb]),
                            plsc.load_gather(tslab_b, [lane16 + 16, cb]))

                return lax.cond(col < TAILA, in_a, in_b)

            write_hits(nh, eout, gather_row_tail)

    return phase1


@functools.lru_cache(maxsize=1)
def _build_phase2():
    @functools.partial(
        pl.kernel,
        mesh=_mesh(),
        compiler_params=pltpu.CompilerParams(needs_layout_passes=False),
        out_type=jax.ShapeDtypeStruct((B,), jnp.float32),
        scratch_types=[
            pltpu.VMEM((BPW * DIM,), jnp.float32),
            pltpu.VMEM((BPW * DIM,), jnp.float32),
            pltpu.VMEM((BPW,), jnp.float32),
        ],
    )
    def phase2(e1_hbm, e2_hbm, out_hbm, e1_v, e2_v, out_v):
        wid = lax.axis_index("s") * NC + lax.axis_index("c")
        base = wid * BPW
        pltpu.sync_copy(e1_hbm.at[pl.ds(base * DIM, BPW * DIM)], e1_v)
        pltpu.sync_copy(e2_hbm.at[pl.ds(base * DIM, BPW * DIM)], e2_v)
        lane32 = lax.iota(jnp.int32, 16) * DIM

        def group_body(g, carry):
            off0 = g * (16 * DIM)
            acc = jnp.zeros((16,), jnp.float32)
            for c in range(DIM):
                offs = lane32 + (off0 + c)
                acc = acc + plsc.load_gather(e1_v, [offs]) * \
                    plsc.load_gather(e2_v, [offs])
            out_v[pl.ds(g * 16, 16)] = _log_sigmoid(acc)
            return carry

        lax.fori_loop(0, BPW // 16, group_body, 0)
        pltpu.sync_copy(out_v, out_hbm.at[pl.ds(base, BPW)])

    return phase2


def kernel(idx1, idx2, emb1, emb2, embs1, embs2):
    del emb1, emb2
    i1 = idx1.astype(jnp.int32)
    i2 = idx2.astype(jnp.int32)
    t1 = embs1.T.reshape(4, 8, N)
    t2 = embs2.T.reshape(4, 8, N)
    tb1 = embs1[TAIL0 + TAILA:].T
    tb2 = embs2[TAIL0 + TAILA:].T
    e1all, e2all = _build_phase1()(i1, i2, t1, t2, tb1, tb2)
    return _build_phase2()(e1all, e2all)


# speedup vs baseline: 5.3638x; 1.0846x over previous
import functools

import jax
import jax.numpy as jnp
from jax import lax
from jax.experimental import pallas as pl
from jax.experimental.pallas import tpu as pltpu
from jax.experimental.pallas import tpu_sc as plsc

N = 1000000
DIM = 32
B = 16384
NC = 2
NS = 16
NW = NC * NS
BPW = B // NW

WL = 1024
NWIN_FULL = N // WL
TAIL0 = NWIN_FULL * WL
TAILA = 512
TAILB = N - TAIL0 - TAILA
WPT = -(-(NWIN_FULL) // NW)
MYCAP = 1024
HITCAP = 256
NSLOT = 16
LAG = 12

_SENTINEL = 0x7FFFFFF0
_LN2 = 0.6931471805599453


def _log_sigmoid(x):
    e = jnp.exp(-x)
    y = 1.0 / (1.0 + e) + 1e-20
    i = lax.bitcast_convert_type(y, jnp.int32)
    ix = i + jnp.int32(0x3F800000 - 0x3F3504F3)
    k = lax.shift_right_arithmetic(ix, jnp.int32(23)) - jnp.int32(127)
    mb = lax.bitwise_and(ix, jnp.int32(0x007FFFFF)) + jnp.int32(0x3F3504F3)
    m = lax.bitcast_convert_type(mb, jnp.float32)
    f = m - 1.0
    s = f / (2.0 + f)
    z = s * s
    w = z * z
    t1 = w * (0.40000972152 + w * 0.24279078841)
    t2 = z * (0.66666662693 + w * 0.28498786688)
    r = t2 + t1
    hfsq = 0.5 * f * f
    kf = k.astype(jnp.float32)
    return kf * _LN2 + (f - (hfsq - s * (hfsq + r)))


def _mesh():
    return plsc.VectorSubcoreMesh(core_axis_name="c", subcore_axis_name="s")


@functools.lru_cache(maxsize=1)
def _build_phase1():
    @functools.partial(
        pl.kernel,
        mesh=_mesh(),
        compiler_params=pltpu.CompilerParams(needs_layout_passes=False),
        out_type=(
            jax.ShapeDtypeStruct((B * DIM,), jnp.float32),
            jax.ShapeDtypeStruct((B * DIM,), jnp.float32),
        ),
        scratch_types=[
            pltpu.VMEM((B,), jnp.int32),
            pltpu.VMEM((B,), jnp.int32),
            pltpu.VMEM((MYCAP,), jnp.int32),
            pltpu.VMEM((MYCAP,), jnp.int32),
            pltpu.VMEM((MYCAP,), jnp.int32),
            pltpu.VMEM((MYCAP,), jnp.int32),
            pltpu.VMEM((HITCAP,), jnp.int32),
            pltpu.VMEM((HITCAP,), jnp.int32),
            pltpu.VMEM((2, 4, 8, WL), jnp.float32),
            pltpu.VMEM((4, 8, TAILA), jnp.float32),
            pltpu.VMEM((DIM, TAILB), jnp.float32),
            pltpu.VMEM((NSLOT, DIM), jnp.float32),
            pltpu.SemaphoreType.DMA,
            pltpu.SemaphoreType.DMA,
        ],
    )
    def phase1(idx1_hbm, idx2_hbm, t1_hbm, t2_hbm, tb1_hbm, tb2_hbm,
               e1_hbm, e2_hbm,
               idx1_v, idx2_v, myk0, myk1, myi0, myi1, hitk, hitc,
               slab, tslab_a, tslab_b, rowstage, sem_s, sem_w):
        wid = lax.axis_index("s") * NC + lax.axis_index("c")
        lane = lax.iota(jnp.int32, 16)
        b0 = lax.shift_right_logical(lane, jnp.int32(3))
        s0 = lax.bitwise_and(lane, jnp.int32(7))
        b1 = b0 + 2

        pltpu.sync_copy(idx1_hbm, idx1_v)
        pltpu.sync_copy(idx2_hbm, idx2_v)

        sent = jnp.full((16,), _SENTINEL, jnp.int32)
        def fill_body(g, carry):
            myi0[pl.ds(g * 16, 16)] = sent
            myi1[pl.ds(g * 16, 16)] = sent
            return carry
        lax.fori_loop(0, MYCAP // 16, fill_body, 0)

        def extract_body(g2, curs):
            cur0, cur1 = curs
            for u in range(2):
                g = g2 * 2 + u
                kv = lane + g * 16
                iv1 = idx1_v[pl.ds(g * 16, 16)]
                w1 = lax.shift_right_logical(iv1, jnp.int32(10))
                m1 = lax.bitwise_and(w1, jnp.int32(NW - 1)) == wid
                plsc.store_compressed(myk0.at[pl.ds(cur0, 16)], kv, mask=m1)
                plsc.store_compressed(myi0.at[pl.ds(cur0, 16)], iv1, mask=m1)
                cur0 = cur0 + plsc.all_reduce_population_count(m1)[0]
                iv2 = idx2_v[pl.ds(g * 16, 16)]
                w2 = lax.shift_right_logical(iv2, jnp.int32(10))
                m2 = lax.bitwise_and(w2, jnp.int32(NW - 1)) == wid
                plsc.store_compressed(myk1.at[pl.ds(cur1, 16)], kv, mask=m2)
                plsc.store_compressed(myi1.at[pl.ds(cur1, 16)], iv2, mask=m2)
                cur1 = cur1 + plsc.all_reduce_population_count(m2)[0]
            return cur0, cur1

        nmine = list(lax.fori_loop(
            0, B // 32, extract_body, (jnp.int32(0), jnp.int32(0))))

        def find_hits(t, w, w0):
            mk = myk0 if t == 0 else myk1
            mi = myi0 if t == 0 else myi1
            ng2 = lax.shift_right_logical(nmine[t] + 31, jnp.int32(5))

            def scan_body(g2, cur):
                for u in range(2):
                    g = g2 * 2 + u
                    iv = mi[pl.ds(g * 16, 16)]
                    m = lax.shift_right_logical(iv, jnp.int32(10)) == w
                    kv = mk[pl.ds(g * 16, 16)]
                    plsc.store_compressed(hitk.at[pl.ds(cur, 16)], kv, mask=m)
                    plsc.store_compressed(
                        hitc.at[pl.ds(cur, 16)], iv - w0, mask=m)
                    cur = cur + plsc.all_reduce_population_count(m)[0]
                return cur
            return lax.fori_loop(0, ng2, scan_body, jnp.int32(0))

        def write_hits(nh, eout, gather_row):
            def hit_body(h, carry):
                kvec = plsc.load_gather(hitk, [jnp.full((16,), h, jnp.int32)])
                cvec = plsc.load_gather(hitc, [jnp.full((16,), h, jnp.int32)])
                k = kvec[0]
                v0, v1 = gather_row(cvec)
                slot = lax.rem(h, jnp.int32(NSLOT))
                rowstage[slot, pl.ds(0, 16)] = v0
                rowstage[slot, pl.ds(16, 16)] = v1
                pltpu.async_copy(rowstage.at[slot],
                                 eout.at[pl.ds(k * DIM, DIM)], sem_w)

                @pl.when(h >= LAG)
                def _():
                    pltpu.make_async_copy(
                        eout.at[pl.ds(0, DIM)], rowstage.at[0], sem_w).wait()

                return carry

            lax.fori_loop(0, nh, hit_body, 0)

            def drain_body(r, carry):
                pltpu.make_async_copy(
                    eout.at[pl.ds(0, DIM)], rowstage.at[0], sem_w).wait()
                return carry

            lax.fori_loop(0, jnp.minimum(nh, jnp.int32(LAG)), drain_body, 0)

        for t in range(2):
            tbl = t1_hbm if t == 0 else t2_hbm
            eout = e1_hbm if t == 0 else e2_hbm

            @pl.when(wid < NWIN_FULL)
            def _(tbl=tbl):
                for b in range(4):
                    pltpu.async_copy(
                        tbl.at[b, :, pl.ds(wid * WL, WL)], slab.at[0, b],
                        sem_s)

            def win_body(n, carry, t=t, tbl=tbl, eout=eout):
                w = wid + n * NW
                buf = lax.rem(n, jnp.int32(2))
                nbuf = lax.rem(n + 1, jnp.int32(2))
                wn = w + NW

                @pl.when(wn < NWIN_FULL)
                def _():
                    for b in range(4):
                        pltpu.async_copy(
                            tbl.at[b, :, pl.ds(wn * WL, WL)],
                            slab.at[nbuf, b], sem_s)

                @pl.when(w < NWIN_FULL)
                def _():
                    w0 = w * WL
                    nh = find_hits(t, w, w0)
                    for b in range(4):
                        pltpu.make_async_copy(
                            tbl.at[b, :, pl.ds(0, WL)], slab.at[0, b],
                            sem_s).wait()
                    bufv = jnp.full((16,), buf, jnp.int32)

                    def gather_row(cvec):
                        return (plsc.load_gather(slab, [bufv, b0, s0, cvec]),
                                plsc.load_gather(slab, [bufv, b1, s0, cvec]))

                    write_hits(nh, eout, gather_row)

                return carry

            lax.fori_loop(0, WPT, win_body, 0)

        for t in range(2):
            tbl = t1_hbm if t == 0 else t2_hbm
            tbt = tb1_hbm if t == 0 else tb2_hbm
            eout = e1_hbm if t == 0 else e2_hbm
            idxv = idx1_v if t == 0 else idx2_v
            for b in range(4):
                pltpu.async_copy(
                    tbl.at[b, :, pl.ds(TAIL0, TAILA)], tslab_a.at[b], sem_s)
            pltpu.sync_copy(tbt, tslab_b)

            def tail_scan(g, cur, idxv=idxv):
                iv = idxv[pl.ds(g * 16, 16)]
                m = iv >= TAIL0
                kv = lane + g * 16
                plsc.store_compressed(hitk.at[pl.ds(cur, 16)], kv, mask=m)
                plsc.store_compressed(hitc.at[pl.ds(cur, 16)], iv - TAIL0, mask=m)
                cnt = plsc.all_reduce_population_count(m)
                return cur + cnt[0]

            nh = lax.fori_loop(0, B // 16, tail_scan, jnp.int32(0))
            for b in range(4):
                pltpu.make_async_copy(
                    tbl.at[b, :, pl.ds(0, TAILA)], tslab_a.at[b], sem_s).wait()

            def gather_row_tail(cvec):
                col = cvec[0]

                def in_a():
                    return (plsc.load_gather(tslab_a, [b0, s0, cvec]),
                            plsc.load_gather(tslab_a, [b1, s0, cvec]))

                def in_b():
                    cb = cvec - TAILA
                    lane16 = b0 * 8 + s0
                    return (plsc.load_gather(tslab_b, [lane16, cb]),
                            plsc.load_gather(tslab_b, [lane16 + 16, cb]))

                return lax.cond(col < TAILA, in_a, in_b)

            write_hits(nh, eout, gather_row_tail)

    return phase1


@functools.lru_cache(maxsize=1)
def _build_phase2():
    @functools.partial(
        pl.kernel,
        mesh=_mesh(),
        compiler_params=pltpu.CompilerParams(needs_layout_passes=False),
        out_type=jax.ShapeDtypeStruct((B,), jnp.float32),
        scratch_types=[
            pltpu.VMEM((BPW * DIM,), jnp.float32),
            pltpu.VMEM((BPW * DIM,), jnp.float32),
            pltpu.VMEM((BPW,), jnp.float32),
        ],
    )
    def phase2(e1_hbm, e2_hbm, out_hbm, e1_v, e2_v, out_v):
        wid = lax.axis_index("s") * NC + lax.axis_index("c")
        base = wid * BPW
        pltpu.sync_copy(e1_hbm.at[pl.ds(base * DIM, BPW * DIM)], e1_v)
        pltpu.sync_copy(e2_hbm.at[pl.ds(base * DIM, BPW * DIM)], e2_v)
        lane = lax.iota(jnp.int32, 16)
        lane32 = lane * DIM

        def group_body(g, carry):
            off0 = g * (16 * DIM)
            acc = jnp.zeros((16,), jnp.float32)
            for c in range(DIM):
                cc = lax.bitwise_and(lane + c, jnp.int32(DIM - 1))
                offs = lane32 + off0 + cc
                acc = acc + plsc.load_gather(e1_v, [offs]) * \
                    plsc.load_gather(e2_v, [offs])
            out_v[pl.ds(g * 16, 16)] = _log_sigmoid(acc)
            return carry

        lax.fori_loop(0, BPW // 16, group_body, 0)
        pltpu.sync_copy(out_v, out_hbm.at[pl.ds(base, BPW)])

    return phase2


def kernel(idx1, idx2, emb1, emb2, embs1, embs2):
    del emb1, emb2
    i1 = idx1.astype(jnp.int32)
    i2 = idx2.astype(jnp.int32)
    t1 = embs1.T.reshape(4, 8, N)
    t2 = embs2.T.reshape(4, 8, N)
    tb1 = embs1[TAIL0 + TAILA:].T
    tb2 = embs2[TAIL0 + TAILA:].T
    e1all, e2all = _build_phase1()(i1, i2, t1, t2, tb1, tb2)
    return _build_phase2()(e1all, e2all)
